# baseline (speedup 1.0000x reference)
import functools

import jax
import jax.numpy as jnp
from jax.experimental import pallas as pl
from jax.experimental.pallas import tpu as pltpu

N_NODES = 10000
N_GRAPHS = 64
BLOCK = 2000
NUM_BLOCKS = N_NODES // BLOCK
NQ = 4
CH = BLOCK // NQ


def _copy(x_hbm, xbuf, sem, block_idx, slot, q):
    return pltpu.make_async_copy(
        x_hbm.at[pl.ds(block_idx * BLOCK + q * CH, CH), :],
        xbuf.at[slot, pl.ds(q * CH, CH), :],
        sem.at[slot, q])


def _fused_body(x_hbm, batch_ref, w1_ref, w2_ref, wend_ref, out_ref,
                xbuf, sums_ref, counts_ref, sem):
    i = pl.program_id(0)

    @pl.when(i == 0)
    def _init():
        sums_ref[...] = jnp.zeros_like(sums_ref)
        counts_ref[...] = jnp.zeros_like(counts_ref)
        for q in range(NQ):
            _copy(x_hbm, xbuf, sem, 0, 0, q).start()

    @pl.when(i + 1 < NUM_BLOCKS)
    def _prefetch():
        for q in range(NQ):
            _copy(x_hbm, xbuf, sem, i + 1, (i + 1) % 2, q).start()

    for q in range(NQ):
        _copy(x_hbm, xbuf, sem, i, i % 2, q).wait()

    x = xbuf[i % 2]
    h = jax.lax.dot(x, w1_ref[...],
                    preferred_element_type=jnp.float32)
    h = jnp.maximum(h, 0.0)
    h = jax.lax.dot(h, w2_ref[...],
                    preferred_element_type=jnp.float32)
    h = jnp.maximum(h, 0.0)
    h3 = jax.lax.dot(h, wend_ref[...],
                     preferred_element_type=jnp.float32)

    b = batch_ref[0]
    seg = jax.lax.broadcasted_iota(jnp.int32, (N_GRAPHS, BLOCK), 0)
    maskf = (b == seg).astype(jnp.float32)
    sums_ref[...] += jax.lax.dot(maskf, h3,
                                 preferred_element_type=jnp.float32)
    counts_ref[...] += jnp.sum(maskf, axis=1, keepdims=True)

    @pl.when(i == NUM_BLOCKS - 1)
    def _fin():
        pooled = sums_ref[...] / jnp.maximum(counts_ref[...], 1.0)
        out_ref[...] = jax.nn.sigmoid(pooled)


@functools.partial(jax.jit, static_argnames=())
def _fused_call(x, batch3, W1, W2, Wend):
    return pl.pallas_call(
        _fused_body,
        grid=(NUM_BLOCKS,),
        in_specs=[
            pl.BlockSpec(memory_space=pl.ANY),
            pl.BlockSpec((1, 1, BLOCK), lambda i: (i, 0, 0)),
            pl.BlockSpec((128, 128), lambda i: (0, 0)),
            pl.BlockSpec((128, 128), lambda i: (0, 0)),
            pl.BlockSpec((128, 1), lambda i: (0, 0)),
        ],
        out_specs=pl.BlockSpec((N_GRAPHS, 1), lambda i: (0, 0)),
        out_shape=jax.ShapeDtypeStruct((N_GRAPHS, 1), jnp.float32),
        scratch_shapes=[
            pltpu.VMEM((2, BLOCK, 128), jnp.float32),
            pltpu.VMEM((N_GRAPHS, 1), jnp.float32),
            pltpu.VMEM((N_GRAPHS, 1), jnp.float32),
            pltpu.SemaphoreType.DMA((2, NQ)),
        ],
        compiler_params=pltpu.CompilerParams(
            dimension_semantics=("arbitrary",),
        ),
    )(x, batch3, W1, W2, Wend)


def kernel(x, edge_index, edge_attr, batch, W1, W2, Wend):
    del edge_index, edge_attr
    batch3 = batch.reshape(NUM_BLOCKS, 1, BLOCK)
    return _fused_call(x, batch3, W1, W2, Wend)


# speedup vs baseline: 1.1144x; 1.1144x over previous
import functools

import jax
import jax.numpy as jnp
from jax.experimental import pallas as pl
from jax.experimental.pallas import tpu as pltpu

N_NODES = 10000
N_GRAPHS = 64
HIDDEN = 128
BLOCK = 2000
NUM_BLOCKS = N_NODES // BLOCK


def _fused_body(x_ref, batch_ref, w1_ref, w2_ref, wend_ref, out_ref,
                acc_ref, counts_ref):
    i = pl.program_id(0)

    @pl.when(i == 0)
    def _init():
        acc_ref[...] = jnp.zeros_like(acc_ref)
        counts_ref[...] = jnp.zeros_like(counts_ref)

    x = x_ref[...]
    h = jax.lax.dot(x, w1_ref[...],
                    preferred_element_type=jnp.float32)
    h = jnp.maximum(h, 0.0)
    h = jax.lax.dot(h, w2_ref[...],
                    preferred_element_type=jnp.float32)
    h = jnp.maximum(h, 0.0)

    b = batch_ref[0]
    seg = jax.lax.broadcasted_iota(jnp.int32, (N_GRAPHS, BLOCK), 0)
    maskf = (b == seg).astype(jnp.float32)
    acc_ref[...] += jax.lax.dot(maskf, h,
                                preferred_element_type=jnp.float32)
    counts_ref[...] += jnp.sum(maskf, axis=1, keepdims=True)

    @pl.when(i == NUM_BLOCKS - 1)
    def _fin():
        sums = jax.lax.dot(acc_ref[...], wend_ref[...],
                           preferred_element_type=jnp.float32)
        pooled = sums / jnp.maximum(counts_ref[...], 1.0)
        out_ref[...] = jax.nn.sigmoid(pooled)


@functools.partial(jax.jit, static_argnames=())
def _fused_call(x, batch3, W1, W2, Wend):
    return pl.pallas_call(
        _fused_body,
        grid=(NUM_BLOCKS,),
        in_specs=[
            pl.BlockSpec((BLOCK, 128), lambda i: (i, 0)),
            pl.BlockSpec((1, 1, BLOCK), lambda i: (i, 0, 0)),
            pl.BlockSpec((128, 128), lambda i: (0, 0)),
            pl.BlockSpec((128, 128), lambda i: (0, 0)),
            pl.BlockSpec((128, 1), lambda i: (0, 0)),
        ],
        out_specs=pl.BlockSpec((N_GRAPHS, 1), lambda i: (0, 0)),
        out_shape=jax.ShapeDtypeStruct((N_GRAPHS, 1), jnp.float32),
        scratch_shapes=[
            pltpu.VMEM((N_GRAPHS, HIDDEN), jnp.float32),
            pltpu.VMEM((N_GRAPHS, 1), jnp.float32),
        ],
        compiler_params=pltpu.CompilerParams(
            dimension_semantics=("arbitrary",),
        ),
    )(x, batch3, W1, W2, Wend)


def kernel(x, edge_index, edge_attr, batch, W1, W2, Wend):
    del edge_index, edge_attr
    batch3 = batch.reshape(NUM_BLOCKS, 1, BLOCK)
    return _fused_call(x, batch3, W1, W2, Wend)


# speedup vs baseline: 1.5053x; 1.3508x over previous
import functools

import jax
import jax.numpy as jnp
from jax.experimental import pallas as pl
from jax.experimental.pallas import tpu as pltpu

N_NODES = 10000
N_GRAPHS = 64
D = 128
CHUNK = 2000
NUM_CHUNKS = N_NODES // CHUNK


def _copy(x_hbm, xbuf, sem, k):
    return pltpu.make_async_copy(
        x_hbm.at[pl.ds(k * CHUNK, CHUNK), :],
        xbuf.at[k % 2],
        sem.at[k % 2])


def _fused_body(x_hbm, batch_ref, w1_ref, w2_ref, wend_ref, out_ref,
                xbuf, sem):
    w1 = w1_ref[...]
    w2 = w2_ref[...]
    wend_row = wend_ref[...].reshape(1, D)
    b_all = batch_ref[...].reshape(1, N_NODES)
    seg = jax.lax.broadcasted_iota(jnp.int32, (N_GRAPHS, CHUNK), 0)

    _copy(x_hbm, xbuf, sem, 0).start()
    acc = jnp.zeros((N_GRAPHS, D), jnp.float32)
    counts = jnp.zeros((N_GRAPHS, 1), jnp.float32)
    for k in range(NUM_CHUNKS):
        if k + 1 < NUM_CHUNKS:
            _copy(x_hbm, xbuf, sem, k + 1).start()
        _copy(x_hbm, xbuf, sem, k).wait()
        x = xbuf[k % 2]
        h = jax.lax.dot(x, w1, preferred_element_type=jnp.float32)
        h = jnp.maximum(h, 0.0)
        h = jax.lax.dot(h, w2, preferred_element_type=jnp.float32)
        h = jnp.maximum(h, 0.0)
        b = jax.lax.slice(b_all, (0, k * CHUNK), (1, (k + 1) * CHUNK))
        maskf = (b == seg).astype(jnp.float32)
        acc = acc + jax.lax.dot(maskf, h, preferred_element_type=jnp.float32)
        counts = counts + jnp.sum(maskf, axis=1, keepdims=True)

    sums = jnp.sum(acc * wend_row, axis=1, keepdims=True)
    pooled = sums / jnp.maximum(counts, 1.0)
    out_ref[...] = jax.nn.sigmoid(pooled)


@functools.partial(jax.jit, static_argnames=())
def _fused_call(x, batch, W1, W2, wend_flat):
    return pl.pallas_call(
        _fused_body,
        in_specs=[
            pl.BlockSpec(memory_space=pl.ANY),
            pl.BlockSpec((N_NODES,), lambda: (0,)),
            pl.BlockSpec((D, D), lambda: (0, 0)),
            pl.BlockSpec((D, D), lambda: (0, 0)),
            pl.BlockSpec((D,), lambda: (0,)),
        ],
        out_specs=pl.BlockSpec((N_GRAPHS, 1), lambda: (0, 0)),
        out_shape=jax.ShapeDtypeStruct((N_GRAPHS, 1), jnp.float32),
        scratch_shapes=[
            pltpu.VMEM((2, CHUNK, D), jnp.float32),
            pltpu.SemaphoreType.DMA((2,)),
        ],
    )(x, batch, W1, W2, wend_flat)


def kernel(x, edge_index, edge_attr, batch, W1, W2, Wend):
    del edge_index, edge_attr
    return _fused_call(x, batch, W1, W2, Wend.reshape(D))


# speedup vs baseline: 1.8303x; 1.2159x over previous
import functools

import jax
import jax.numpy as jnp
from jax.experimental import pallas as pl
from jax.experimental.pallas import tpu as pltpu

N_NODES = 10000
N_GRAPHS = 64
D = 128
CHUNK = 2000
NUM_CHUNKS = N_NODES // CHUNK


def _copy(x_hbm, xbuf, sem, k):
    return pltpu.make_async_copy(
        x_hbm.at[pl.ds(k * CHUNK, CHUNK), :],
        xbuf.at[k % 2],
        sem.at[k % 2])


def _fused_body(x_hbm, batch_ref, w1_ref, w2_ref, wend_ref, out_ref,
                xbuf, sem):
    w1 = w1_ref[...]
    w2 = w2_ref[...]
    wend_row = wend_ref[...].reshape(1, D)
    b_all = batch_ref[...].reshape(1, N_NODES)
    seg = jax.lax.broadcasted_iota(jnp.int32, (N_GRAPHS, CHUNK), 0)

    ones_row = jnp.ones((1, CHUNK), jnp.float32)
    _copy(x_hbm, xbuf, sem, 0).start()
    acc = jnp.zeros((N_GRAPHS, D), jnp.float32)
    counts = jnp.zeros((1, N_GRAPHS), jnp.float32)
    for k in range(NUM_CHUNKS):
        if k + 1 < NUM_CHUNKS:
            _copy(x_hbm, xbuf, sem, k + 1).start()
        _copy(x_hbm, xbuf, sem, k).wait()
        x = xbuf[k % 2]
        h = jax.lax.dot(x, w1, preferred_element_type=jnp.float32)
        h = jnp.maximum(h, 0.0)
        h = jax.lax.dot(h, w2, preferred_element_type=jnp.float32)
        h = jnp.maximum(h, 0.0)
        b = jax.lax.slice(b_all, (0, k * CHUNK), (1, (k + 1) * CHUNK))
        maskf = (b == seg).astype(jnp.float32)
        acc = acc + jax.lax.dot(maskf, h, preferred_element_type=jnp.float32)
        counts = counts + jax.lax.dot_general(
            ones_row, maskf, (((1,), (1,)), ((), ())),
            preferred_element_type=jnp.float32)

    sums = jax.lax.dot_general(
        wend_row, acc, (((1,), (1,)), ((), ())),
        preferred_element_type=jnp.float32)
    pooled = sums / jnp.maximum(counts, 1.0)
    out_ref[...] = jax.nn.sigmoid(pooled).reshape(N_GRAPHS)


@functools.partial(jax.jit, static_argnames=())
def _fused_call(x, batch, W1, W2, wend_flat):
    return pl.pallas_call(
        _fused_body,
        in_specs=[
            pl.BlockSpec(memory_space=pl.ANY),
            pl.BlockSpec((N_NODES,), lambda: (0,)),
            pl.BlockSpec((D, D), lambda: (0, 0)),
            pl.BlockSpec((D, D), lambda: (0, 0)),
            pl.BlockSpec((D,), lambda: (0,)),
        ],
        out_specs=pl.BlockSpec((N_GRAPHS,), lambda: (0,)),
        out_shape=jax.ShapeDtypeStruct((N_GRAPHS,), jnp.float32),
        scratch_shapes=[
            pltpu.VMEM((2, CHUNK, D), jnp.float32),
            pltpu.SemaphoreType.DMA((2,)),
        ],
    )(x, batch, W1, W2, wend_flat)


def kernel(x, edge_index, edge_attr, batch, W1, W2, Wend):
    del edge_index, edge_attr
    return _fused_call(x, batch, W1, W2, Wend.reshape(D)).reshape(N_GRAPHS, 1)
